# baseline (speedup 1.0000x reference)
import functools

import jax
import jax.numpy as jnp
from jax import lax
from jax.experimental import pallas as pl
from jax.experimental.pallas import tpu as pltpu
from jax.experimental.pallas import tpu_sc as plsc

BM = 128
L = 16


def _router_body(x_ref, nw_ref, wr_ref, xn_ref, ei_ref, wp_ref, *, n_experts):
    x = x_ref[...]
    var = jnp.mean(jnp.square(x), axis=-1, keepdims=True)
    xn = x * jax.lax.rsqrt(var + 1e-6) * nw_ref[...]
    xn_ref[...] = xn
    logits = jnp.dot(xn, wr_ref[...], preferred_element_type=jnp.float32)
    probs = jax.nn.softmax(logits, axis=-1)
    lane = jax.lax.broadcasted_iota(jnp.int32, probs.shape, 1)
    m1 = jnp.max(probs, axis=-1, keepdims=True)
    i1 = jnp.min(jnp.where(probs == m1, lane, n_experts), axis=-1, keepdims=True)
    pm = jnp.where(lane == i1, -jnp.inf, probs)
    m2 = jnp.max(pm, axis=-1, keepdims=True)
    i2 = jnp.min(jnp.where(pm == m2, lane, n_experts), axis=-1, keepdims=True)
    ei_ref[...] = jnp.concatenate([i1, i2], axis=1)
    wp_ref[...] = jnp.concatenate([m1, m2], axis=1)


def _router(x2, nw2, W_r):
    t, d = x2.shape
    e = W_r.shape[1]
    return pl.pallas_call(
        functools.partial(_router_body, n_experts=e),
        grid=(1,),
        in_specs=[
            pl.BlockSpec((t, d), lambda i: (0, 0)),
            pl.BlockSpec((1, d), lambda i: (0, 0)),
            pl.BlockSpec((d, e), lambda i: (0, 0)),
        ],
        out_specs=[
            pl.BlockSpec((t, d), lambda i: (0, 0)),
            pl.BlockSpec((t, 2), lambda i: (0, 0)),
            pl.BlockSpec((t, 2), lambda i: (0, 0)),
        ],
        out_shape=[
            jax.ShapeDtypeStruct((t, d), jnp.float32),
            jax.ShapeDtypeStruct((t, 2), jnp.int32),
            jax.ShapeDtypeStruct((t, 2), jnp.float32),
        ],
    )(x2, nw2, W_r)


def _lanes():
    return lax.iota(jnp.int32, L)


def _g(x, idx):
    return x.at[idx].get(mode="promise_in_bounds")


def _bsum(x):
    lanes = _lanes()
    for k in (1, 2, 4, 8):
        x = x + _g(x, lanes ^ k)
    return x


def _prefix_incl(x):
    lanes = _lanes()
    for k in (1, 2, 4, 8):
        sh = _g(x, jnp.maximum(lanes - k, 0))
        x = x + jnp.where(lanes >= k, sh, 0)
    return x


def _splat(x, e):
    return _g(x, jnp.full((L,), e, jnp.int32))


def _sort_body(n_pairs, m_pad, n_experts, n_tiles_max, dump_tile,
               eid_hbm, wp_hbm,
               pos_hbm, tok_hbm, ws_hbm, st_hbm, fl_hbm,
               eid_v, wv, pos_v, tok_v, dst_v, myhist, histall, hist_sh,
               zvi, zvf, stv, flv, sem):
    c = lax.axis_index("c")
    s = lax.axis_index("s")
    per_w = n_pairs // L
    pad_w = m_pad // L
    n_v = per_w // L
    lanes = lax.iota(jnp.int32, L)
    ones = jnp.ones((L,), jnp.int32)
    zeros = jnp.zeros((L,), jnp.int32)

    def _vcount(v, acc):
        def e_body(e, a):
            cnt = _bsum(jnp.where(v == e, 1, 0))
            return a + jnp.where(lanes == e, cnt, 0)
        return lax.fori_loop(0, n_experts, e_body, acc)

    @pl.when(c == 0)
    def _():
        def z_body(j, _):
            zvi[pl.ds(j * L, L)] = zeros
            zvf[pl.ds(j * L, L)] = jnp.zeros((L,), jnp.float32)
            return _
        lax.fori_loop(0, pad_w // L, z_body, 0)
        pltpu.sync_copy(zvi, tok_hbm.at[pl.ds(s * pad_w, pad_w)])
        pltpu.sync_copy(zvf, ws_hbm.at[pl.ds(s * pad_w, pad_w)])

        pltpu.sync_copy(eid_hbm.at[pl.ds(s * per_w, per_w)], eid_v)
        pltpu.sync_copy(wp_hbm.at[pl.ds(s * per_w, per_w)], wv)

        def h_body(j, hist):
            return _vcount(eid_v[pl.ds(j * L, L)], hist)
        myhist[...] = lax.fori_loop(0, n_v, h_body, zeros)
        pltpu.sync_copy(myhist, hist_sh.at[s])
        plsc.subcore_barrier()

        pltpu.sync_copy(hist_sh, histall)

        def t_body(j, acc):
            return acc + histall[j, :]
        totals = lax.fori_loop(0, L, t_body, zeros)
        before = lax.fori_loop(0, s, t_body, zeros)
        padded = ((totals + (BM - 1)) >> 7) << 7
        incl = _prefix_incl(padded)
        base = incl - padded

        @pl.when(s == 0)
        def _():
            base_t = base >> 7
            nt_t = padded >> 7

            def s_body(e, _):
                bt = _splat(base_t, e)
                nt = _splat(nt_t, e)
                m = bt + jnp.minimum(lanes, jnp.maximum(nt - 1, 0))
                flag = jnp.where(lanes < nt, 1, 0)
                m = jnp.where(flag == 1, jnp.minimum(m, dump_tile), dump_tile)
                stv[pl.ds(e * L, L)] = m
                flv[pl.ds(e * L, L)] = flag
                return _
            lax.fori_loop(0, n_experts, s_body, 0)
            pltpu.sync_copy(stv, st_hbm)
            pltpu.sync_copy(flv, fl_hbm)

        p_base = s * per_w
        n_tok = n_pairs // 2

        def pos_body(j, off):
            v = eid_v[pl.ds(j * L, L)]
            off_l = _g(off, v)

            def k_body(k, r):
                sh = _g(v, jnp.maximum(lanes - k, 0))
                return r + jnp.where((sh == v) & (lanes >= k), 1, 0)
            rank = lax.fori_loop(1, L, k_body, zeros)
            pos_v[pl.ds(j * L, L)] = off_l + rank
            p_vec = p_base + j * L + lanes
            tok_v[pl.ds(j * L, L)] = p_vec >> 1
            dst_v[pl.ds(j * L, L)] = (p_vec & 1) * n_tok + (p_vec >> 1)
            return _vcount(v, off)
        lax.fori_loop(0, n_v, pos_body, base + before)

        pltpu.async_copy(pos_v, pos_hbm.at[dst_v], sem).wait()
        pltpu.async_copy(tok_v, tok_hbm.at[pos_v], sem).wait()
        pltpu.async_copy(wv, ws_hbm.at[pos_v], sem).wait()


def _sort(eflat, wflat, m_pad, n_experts, n_tiles_max, dump_tile):
    n_pairs = eflat.shape[0]
    per_w = n_pairs // L
    pad_w = m_pad // L
    n_steps = n_experts * n_tiles_max
    mesh = plsc.VectorSubcoreMesh(core_axis_name="c", subcore_axis_name="s", num_cores=2, num_subcores=16)
    f = pl.kernel(
        functools.partial(_sort_body, n_pairs, m_pad, n_experts, n_tiles_max,
                          dump_tile),
        out_type=[
            jax.ShapeDtypeStruct((n_pairs,), jnp.int32),
            jax.ShapeDtypeStruct((m_pad,), jnp.int32),
            jax.ShapeDtypeStruct((m_pad,), jnp.float32),
            jax.ShapeDtypeStruct((n_steps,), jnp.int32),
            jax.ShapeDtypeStruct((n_steps,), jnp.int32),
        ],
        mesh=mesh,
        scratch_types=[
            pltpu.VMEM((per_w,), jnp.int32),
            pltpu.VMEM((per_w,), jnp.float32),
            pltpu.VMEM((per_w,), jnp.int32),
            pltpu.VMEM((per_w,), jnp.int32),
            pltpu.VMEM((per_w,), jnp.int32),
            pltpu.VMEM((L,), jnp.int32),
            pltpu.VMEM((L, L), jnp.int32),
            pltpu.VMEM_SHARED((L, L), jnp.int32),
            pltpu.VMEM((pad_w,), jnp.int32),
            pltpu.VMEM((pad_w,), jnp.float32),
            pltpu.VMEM((n_steps,), jnp.int32),
            pltpu.VMEM((n_steps,), jnp.int32),
            pltpu.SemaphoreType.DMA,
        ],
    )
    return f(eflat, wflat)


def _gather_body(rows_w, chunk, nc, tok_hbm, xn_hbm, xp_hbm, idx_v, buf, sem):
    c = lax.axis_index("c")
    s = lax.axis_index("s")
    wid = s * nc + c
    base = wid * rows_w

    def step(ci, _):
        r0 = base + ci * chunk
        pltpu.sync_copy(tok_hbm.at[pl.ds(r0, chunk)], idx_v)
        pltpu.async_copy(xn_hbm.at[idx_v], buf, sem).wait()
        pltpu.sync_copy(buf, xp_hbm.at[pl.ds(r0, chunk)])
        return _

    lax.fori_loop(0, rows_w // chunk, step, 0)


def _gather(tok, xn, m_pad):
    t, d = xn.shape
    mesh = plsc.VectorSubcoreMesh(core_axis_name="c", subcore_axis_name="s", num_cores=2, num_subcores=16)
    nw = mesh.num_cores * mesh.num_subcores
    rows_w = m_pad // nw
    chunk = 24
    f = pl.kernel(
        functools.partial(_gather_body, rows_w, chunk, mesh.num_cores),
        out_type=jax.ShapeDtypeStruct((m_pad, d), jnp.float32),
        mesh=mesh,
        scratch_types=[
            pltpu.VMEM((chunk,), jnp.int32),
            pltpu.VMEM((chunk, d), jnp.float32),
            pltpu.SemaphoreType.DMA,
        ],
    )
    return f(tok, xn)


def _k1_body(st_ref, fl_ref, x_ref, wg_ref, wu_ref, ws_ref, h_ref,
             wg_bf, wu_bf, *, n_tiles_max):
    e = pl.program_id(0)
    j = pl.program_id(1)
    flag = fl_ref[e * n_tiles_max + j] == 1

    @pl.when(flag & (j == 0))
    def _():
        wg_bf[...] = wg_ref[0].astype(jnp.bfloat16)
        wu_bf[...] = wu_ref[0].astype(jnp.bfloat16)

    @pl.when(flag)
    def _():
        xb = x_ref[...].astype(jnp.bfloat16)
        gate = jnp.dot(xb, wg_bf[...], preferred_element_type=jnp.float32)
        up = jnp.dot(xb, wu_bf[...], preferred_element_type=jnp.float32)
        h = gate * jax.lax.logistic(gate) * up * ws_ref[...]
        h_ref[...] = h.astype(jnp.bfloat16)


def _k1(steptile, flags, xpad, W_gate, W_up, ws2, n_tiles_max):
    m_pad, d = xpad.shape
    n_experts, _, d_exp = W_gate.shape
    grid_spec = pltpu.PrefetchScalarGridSpec(
        num_scalar_prefetch=2,
        grid=(n_experts, n_tiles_max),
        in_specs=[
            pl.BlockSpec((BM, d), lambda e, j, st, fl: (st[e * n_tiles_max + j], 0)),
            pl.BlockSpec((1, d, d_exp), lambda e, j, st, fl: (e, 0, 0)),
            pl.BlockSpec((1, d, d_exp), lambda e, j, st, fl: (e, 0, 0)),
            pl.BlockSpec((BM, 1), lambda e, j, st, fl: (st[e * n_tiles_max + j], 0)),
        ],
        out_specs=pl.BlockSpec((BM, d_exp),
                               lambda e, j, st, fl: (st[e * n_tiles_max + j], 0)),
        scratch_shapes=[
            pltpu.VMEM((d, d_exp), jnp.bfloat16),
            pltpu.VMEM((d, d_exp), jnp.bfloat16),
        ],
    )
    return pl.pallas_call(
        functools.partial(_k1_body, n_tiles_max=n_tiles_max),
        grid_spec=grid_spec,
        out_shape=jax.ShapeDtypeStruct((m_pad, d_exp), jnp.bfloat16),
    )(steptile, flags, xpad, W_gate, W_up, ws2)


def _k2_body(st_ref, fl_ref, h_ref, wd_ref, y_ref, wd_bf, *, n_tiles_max):
    e = pl.program_id(0)
    j = pl.program_id(1)
    flag = fl_ref[e * n_tiles_max + j] == 1

    @pl.when(flag & (j == 0))
    def _():
        wd_bf[...] = wd_ref[0].astype(jnp.bfloat16)

    @pl.when(flag)
    def _():
        y_ref[...] = jnp.dot(h_ref[...], wd_bf[...],
                             preferred_element_type=jnp.float32)


def _k2(steptile, flags, h, W_down, n_tiles_max):
    m_pad, d_exp = h.shape
    n_experts, _, d = W_down.shape
    grid_spec = pltpu.PrefetchScalarGridSpec(
        num_scalar_prefetch=2,
        grid=(n_experts, n_tiles_max),
        in_specs=[
            pl.BlockSpec((BM, d_exp), lambda e, j, st, fl: (st[e * n_tiles_max + j], 0)),
            pl.BlockSpec((1, d_exp, d), lambda e, j, st, fl: (e, 0, 0)),
        ],
        out_specs=pl.BlockSpec((BM, d),
                               lambda e, j, st, fl: (st[e * n_tiles_max + j], 0)),
        scratch_shapes=[pltpu.VMEM((d_exp, d), jnp.bfloat16)],
    )
    return pl.pallas_call(
        functools.partial(_k2_body, n_tiles_max=n_tiles_max),
        grid_spec=grid_spec,
        out_shape=jax.ShapeDtypeStruct((m_pad, d), jnp.float32),
    )(steptile, flags, h, W_down)


def _combine_body(tok_w, chunk, nc, n_tok, x_hbm, y_hbm, pos_hbm, out_hbm,
                  p0_v, p1_v, y0, y1, xb, ob, sem):
    c = lax.axis_index("c")
    s = lax.axis_index("s")
    wid = s * nc + c
    d = xb.shape[1]
    nv = d // L

    def step(ci, _):
        t0 = wid * tok_w + ci * chunk
        pltpu.sync_copy(pos_hbm.at[pl.ds(t0, chunk)], p0_v)
        pltpu.sync_copy(pos_hbm.at[pl.ds(n_tok + t0, chunk)], p1_v)
        cp0 = pltpu.async_copy(y_hbm.at[p0_v], y0, sem)
        cp0.wait()
        cp1 = pltpu.async_copy(y_hbm.at[p1_v], y1, sem)
        cp1.wait()
        pltpu.sync_copy(x_hbm.at[pl.ds(t0, chunk)], xb)

        def row(i, _):
            r = i >> 6
            col = (i & (nv - 1)) * L
            ob[r, pl.ds(col, L)] = (xb[r, pl.ds(col, L)]
                                    + y0[r, pl.ds(col, L)]
                                    + y1[r, pl.ds(col, L)])
            return _

        lax.fori_loop(0, chunk * nv, row, 0)
        pltpu.sync_copy(ob, out_hbm.at[pl.ds(t0, chunk)])
        return _

    lax.fori_loop(0, tok_w // chunk, step, 0)


def _combine(x2, y, pos):
    t, d = x2.shape
    mesh = plsc.VectorSubcoreMesh(core_axis_name="c", subcore_axis_name="s", num_cores=2, num_subcores=16)
    nw = mesh.num_cores * mesh.num_subcores
    tok_w = t // nw
    chunk = 16
    f = pl.kernel(
        functools.partial(_combine_body, tok_w, chunk, mesh.num_cores, t),
        out_type=jax.ShapeDtypeStruct((t, d), jnp.float32),
        mesh=mesh,
        scratch_types=[
            pltpu.VMEM((chunk,), jnp.int32),
            pltpu.VMEM((chunk,), jnp.int32),
            pltpu.VMEM((chunk, d), jnp.float32),
            pltpu.VMEM((chunk, d), jnp.float32),
            pltpu.VMEM((chunk, d), jnp.float32),
            pltpu.VMEM((chunk, d), jnp.float32),
            pltpu.SemaphoreType.DMA,
        ],
    )
    return f(x2, y, pos)


def kernel(x, norm_w, W_r, W_gate, W_up, W_down):
    orig_shape = x.shape
    d = x.shape[-1]
    t = x.size // d
    n_experts = W_gate.shape[0]
    n_tiles_max = t // BM
    m_pad = 2 * t + n_experts * BM + 2 * BM
    dump_tile = m_pad // BM - 2

    x2 = x.reshape(t, d)
    nw2 = norm_w.reshape(1, d)
    xn, ei, wp = _router(x2, nw2, W_r)
    pos, tok, wsort, steptile, flags = _sort(
        ei.reshape(2 * t), wp.reshape(2 * t), m_pad, n_experts,
        n_tiles_max, dump_tile)
    xpad = _gather(tok, xn, m_pad)
    h = _k1(steptile, flags, xpad, W_gate, W_up, wsort.reshape(m_pad, 1),
            n_tiles_max)
    y = _k2(steptile, flags, h, W_down, n_tiles_max)
    out = _combine(x2, y, pos)
    return out.reshape(orig_shape)


# speedup vs baseline: 1.0213x; 1.0213x over previous
import functools

import jax
import jax.numpy as jnp
from jax import lax
from jax.experimental import pallas as pl
from jax.experimental.pallas import tpu as pltpu
from jax.experimental.pallas import tpu_sc as plsc

BM = 128
L = 16


def _router_body(x_ref, nw_ref, wr_ref, xn_ref, ei_ref, wp_ref, *, n_experts):
    x = x_ref[...]
    var = jnp.mean(jnp.square(x), axis=-1, keepdims=True)
    xn = x * jax.lax.rsqrt(var + 1e-6) * nw_ref[...]
    xn_ref[...] = xn
    logits = jnp.dot(xn, wr_ref[...], preferred_element_type=jnp.float32)
    probs = jax.nn.softmax(logits, axis=-1)
    lane = jax.lax.broadcasted_iota(jnp.int32, probs.shape, 1)
    m1 = jnp.max(probs, axis=-1, keepdims=True)
    i1 = jnp.min(jnp.where(probs == m1, lane, n_experts), axis=-1, keepdims=True)
    pm = jnp.where(lane == i1, -jnp.inf, probs)
    m2 = jnp.max(pm, axis=-1, keepdims=True)
    i2 = jnp.min(jnp.where(pm == m2, lane, n_experts), axis=-1, keepdims=True)
    ei_ref[...] = jnp.concatenate([i1, i2], axis=1)
    wp_ref[...] = jnp.concatenate([m1, m2], axis=1)


def _router(x2, nw2, W_r):
    t, d = x2.shape
    e = W_r.shape[1]
    return pl.pallas_call(
        functools.partial(_router_body, n_experts=e),
        grid=(1,),
        in_specs=[
            pl.BlockSpec((t, d), lambda i: (0, 0)),
            pl.BlockSpec((1, d), lambda i: (0, 0)),
            pl.BlockSpec((d, e), lambda i: (0, 0)),
        ],
        out_specs=[
            pl.BlockSpec((t, d), lambda i: (0, 0)),
            pl.BlockSpec((t, 2), lambda i: (0, 0)),
            pl.BlockSpec((t, 2), lambda i: (0, 0)),
        ],
        out_shape=[
            jax.ShapeDtypeStruct((t, d), jnp.float32),
            jax.ShapeDtypeStruct((t, 2), jnp.int32),
            jax.ShapeDtypeStruct((t, 2), jnp.float32),
        ],
    )(x2, nw2, W_r)


def _lanes():
    return lax.iota(jnp.int32, L)


def _g(x, idx):
    return x.at[idx].get(mode="promise_in_bounds")


def _bsum(x):
    lanes = _lanes()
    for k in (1, 2, 4, 8):
        x = x + _g(x, lanes ^ k)
    return x


def _prefix_incl(x):
    lanes = _lanes()
    for k in (1, 2, 4, 8):
        sh = _g(x, jnp.maximum(lanes - k, 0))
        x = x + jnp.where(lanes >= k, sh, 0)
    return x


def _splat(x, e):
    return _g(x, jnp.full((L,), e, jnp.int32))


def _sort_body(n_pairs, m_pad, n_experts, n_tiles_max, dump_tile,
               eid_hbm, wp_hbm,
               pos_hbm, tok_hbm, ws_hbm, st_hbm, fl_hbm,
               eid_v, wv, pos_v, tok_v, dst_v, myhist, histall, hist_sh,
               zvi, zvf, stv, flv, sem):
    c = lax.axis_index("c")
    s = lax.axis_index("s")
    per_w = n_pairs // L
    pad_w = m_pad // L
    n_v = per_w // L
    lanes = lax.iota(jnp.int32, L)
    ones = jnp.ones((L,), jnp.int32)
    zeros = jnp.zeros((L,), jnp.int32)

    def _vcount(v, acc):
        lo = jnp.where(v < 4, jnp.where(v == 0, 1, 0) + jnp.where(v == 1, 1 << 8, 0)
                       + jnp.where(v == 2, 1 << 16, 0) + jnp.where(v == 3, 1 << 24, 0), 0)
        hi = jnp.where(v >= 4, jnp.where(v == 4, 1, 0) + jnp.where(v == 5, 1 << 8, 0)
                       + jnp.where(v == 6, 1 << 16, 0) + jnp.where(v == 7, 1 << 24, 0), 0)
        lo_t = _bsum(lo)
        hi_t = _bsum(hi)
        sel = jnp.where(lanes < 4, lo_t, hi_t)
        sh = (lanes & 3) << 3
        cnt = jax.lax.shift_right_logical(sel, sh) & 255
        return acc + jnp.where(lanes < n_experts, cnt, 0)

    @pl.when(c == 0)
    def _():
        for j in range(pad_w // L):
            zvi[pl.ds(j * L, L)] = zeros
            zvf[pl.ds(j * L, L)] = jnp.zeros((L,), jnp.float32)
        pltpu.sync_copy(zvi, tok_hbm.at[pl.ds(s * pad_w, pad_w)])
        pltpu.sync_copy(zvf, ws_hbm.at[pl.ds(s * pad_w, pad_w)])

        pltpu.sync_copy(eid_hbm.at[pl.ds(s * per_w, per_w)], eid_v)
        pltpu.sync_copy(wp_hbm.at[pl.ds(s * per_w, per_w)], wv)

        hist = zeros
        for j in range(n_v):
            hist = _vcount(eid_v[pl.ds(j * L, L)], hist)
        myhist[...] = hist
        pltpu.sync_copy(myhist, hist_sh.at[s])
        plsc.subcore_barrier()

        pltpu.sync_copy(hist_sh, histall)

        totals = zeros
        for j in range(L):
            totals = totals + histall[j, :]

        def t_body(j, acc):
            return acc + histall[j, :]
        before = lax.fori_loop(0, s, t_body, zeros)
        padded = ((totals + (BM - 1)) >> 7) << 7
        incl = _prefix_incl(padded)
        base = incl - padded

        @pl.when(s == 0)
        def _():
            base_t = base >> 7
            nt_t = padded >> 7
            for e in range(n_experts):
                bt = _splat(base_t, e)
                nt = _splat(nt_t, e)
                m = bt + jnp.minimum(lanes, jnp.maximum(nt - 1, 0))
                flag = jnp.where(lanes < nt, 1, 0)
                m = jnp.where(flag == 1, jnp.minimum(m, dump_tile), dump_tile)
                stv[pl.ds(e * L, L)] = m
                flv[pl.ds(e * L, L)] = flag
            pltpu.sync_copy(stv, st_hbm)
            pltpu.sync_copy(flv, fl_hbm)

        p_base = s * per_w
        n_tok = n_pairs // 2

        off = base + before
        for j in range(n_v):
            v = eid_v[pl.ds(j * L, L)]
            off_l = _g(off, v)
            rank = zeros
            for k in range(1, L):
                sh = _g(v, jnp.maximum(lanes - k, 0))
                rank = rank + jnp.where((sh == v) & (lanes >= k), 1, 0)
            pos_v[pl.ds(j * L, L)] = off_l + rank
            p_vec = p_base + j * L + lanes
            tok_v[pl.ds(j * L, L)] = p_vec >> 1
            dst_v[pl.ds(j * L, L)] = (p_vec & 1) * n_tok + (p_vec >> 1)
            off = _vcount(v, off)

        pltpu.async_copy(pos_v, pos_hbm.at[dst_v], sem).wait()
        pltpu.async_copy(tok_v, tok_hbm.at[pos_v], sem).wait()
        pltpu.async_copy(wv, ws_hbm.at[pos_v], sem).wait()


def _sort(eflat, wflat, m_pad, n_experts, n_tiles_max, dump_tile):
    n_pairs = eflat.shape[0]
    per_w = n_pairs // L
    pad_w = m_pad // L
    n_steps = n_experts * n_tiles_max
    mesh = plsc.VectorSubcoreMesh(core_axis_name="c", subcore_axis_name="s", num_cores=2, num_subcores=16)
    f = pl.kernel(
        functools.partial(_sort_body, n_pairs, m_pad, n_experts, n_tiles_max,
                          dump_tile),
        out_type=[
            jax.ShapeDtypeStruct((n_pairs,), jnp.int32),
            jax.ShapeDtypeStruct((m_pad,), jnp.int32),
            jax.ShapeDtypeStruct((m_pad,), jnp.float32),
            jax.ShapeDtypeStruct((n_steps,), jnp.int32),
            jax.ShapeDtypeStruct((n_steps,), jnp.int32),
        ],
        mesh=mesh,
        scratch_types=[
            pltpu.VMEM((per_w,), jnp.int32),
            pltpu.VMEM((per_w,), jnp.float32),
            pltpu.VMEM((per_w,), jnp.int32),
            pltpu.VMEM((per_w,), jnp.int32),
            pltpu.VMEM((per_w,), jnp.int32),
            pltpu.VMEM((L,), jnp.int32),
            pltpu.VMEM((L, L), jnp.int32),
            pltpu.VMEM_SHARED((L, L), jnp.int32),
            pltpu.VMEM((pad_w,), jnp.int32),
            pltpu.VMEM((pad_w,), jnp.float32),
            pltpu.VMEM((n_steps,), jnp.int32),
            pltpu.VMEM((n_steps,), jnp.int32),
            pltpu.SemaphoreType.DMA,
        ],
    )
    return f(eflat, wflat)


def _gather_body(rows_w, chunk, nc, tok_hbm, xn_hbm, xp_hbm,
                 idx0, idx1, b0, b1, s0, s1):
    c = lax.axis_index("c")
    s = lax.axis_index("s")
    wid = s * nc + c
    base = wid * rows_w
    nch = rows_w // chunk
    idx = (idx0, idx1)
    buf = (b0, b1)
    sem = (s0, s1)

    pltpu.sync_copy(tok_hbm.at[pl.ds(base, chunk)], idx0)
    cps = {0: pltpu.async_copy(xn_hbm.at[idx0], b0, s0)}
    for ci in range(nch):
        p = ci & 1
        q = (ci + 1) & 1
        if ci + 1 < nch:
            r1 = base + (ci + 1) * chunk
            pltpu.sync_copy(tok_hbm.at[pl.ds(r1, chunk)], idx[q])
            cps[ci + 1] = pltpu.async_copy(xn_hbm.at[idx[q]], buf[q], sem[q])
        cps[ci].wait()
        pltpu.sync_copy(buf[p], xp_hbm.at[pl.ds(base + ci * chunk, chunk)])


def _gather(tok, xn, m_pad):
    t, d = xn.shape
    mesh = plsc.VectorSubcoreMesh(core_axis_name="c", subcore_axis_name="s", num_cores=2, num_subcores=16)
    nw = mesh.num_cores * mesh.num_subcores
    rows_w = m_pad // nw
    chunk = 24
    f = pl.kernel(
        functools.partial(_gather_body, rows_w, chunk, mesh.num_cores),
        out_type=jax.ShapeDtypeStruct((m_pad, d), jnp.float32),
        mesh=mesh,
        scratch_types=[
            pltpu.VMEM((chunk,), jnp.int32),
            pltpu.VMEM((chunk,), jnp.int32),
            pltpu.VMEM((chunk, d), jnp.float32),
            pltpu.VMEM((chunk, d), jnp.float32),
            pltpu.SemaphoreType.DMA,
            pltpu.SemaphoreType.DMA,
        ],
    )
    return f(tok, xn)


def _k1_body(st_ref, fl_ref, x_ref, wg_ref, wu_ref, ws_ref, h_ref,
             wg_bf, wu_bf, *, n_tiles_max):
    e = pl.program_id(0)
    j = pl.program_id(1)
    flag = fl_ref[e * n_tiles_max + j] == 1

    @pl.when(flag & (j == 0))
    def _():
        wg_bf[...] = wg_ref[0].astype(jnp.bfloat16)
        wu_bf[...] = wu_ref[0].astype(jnp.bfloat16)

    @pl.when(flag)
    def _():
        xb = x_ref[...].astype(jnp.bfloat16)
        gate = jnp.dot(xb, wg_bf[...], preferred_element_type=jnp.float32)
        up = jnp.dot(xb, wu_bf[...], preferred_element_type=jnp.float32)
        h = gate * jax.lax.logistic(gate) * up * ws_ref[...]
        h_ref[...] = h.astype(jnp.bfloat16)


def _k1(steptile, flags, xpad, W_gate, W_up, ws2, n_tiles_max):
    m_pad, d = xpad.shape
    n_experts, _, d_exp = W_gate.shape
    grid_spec = pltpu.PrefetchScalarGridSpec(
        num_scalar_prefetch=2,
        grid=(n_experts, n_tiles_max),
        in_specs=[
            pl.BlockSpec((BM, d), lambda e, j, st, fl: (st[e * n_tiles_max + j], 0)),
            pl.BlockSpec((1, d, d_exp), lambda e, j, st, fl: (e, 0, 0)),
            pl.BlockSpec((1, d, d_exp), lambda e, j, st, fl: (e, 0, 0)),
            pl.BlockSpec((BM, 1), lambda e, j, st, fl: (st[e * n_tiles_max + j], 0)),
        ],
        out_specs=pl.BlockSpec((BM, d_exp),
                               lambda e, j, st, fl: (st[e * n_tiles_max + j], 0)),
        scratch_shapes=[
            pltpu.VMEM((d, d_exp), jnp.bfloat16),
            pltpu.VMEM((d, d_exp), jnp.bfloat16),
        ],
    )
    return pl.pallas_call(
        functools.partial(_k1_body, n_tiles_max=n_tiles_max),
        grid_spec=grid_spec,
        out_shape=jax.ShapeDtypeStruct((m_pad, d_exp), jnp.bfloat16),
    )(steptile, flags, xpad, W_gate, W_up, ws2)


def _k2_body(st_ref, fl_ref, h_ref, wd_ref, y_ref, wd_bf, *, n_tiles_max):
    e = pl.program_id(0)
    j = pl.program_id(1)
    flag = fl_ref[e * n_tiles_max + j] == 1

    @pl.when(flag & (j == 0))
    def _():
        wd_bf[...] = wd_ref[0].astype(jnp.bfloat16)

    @pl.when(flag)
    def _():
        y_ref[...] = jnp.dot(h_ref[...], wd_bf[...],
                             preferred_element_type=jnp.float32)


def _k2(steptile, flags, h, W_down, n_tiles_max):
    m_pad, d_exp = h.shape
    n_experts, _, d = W_down.shape
    grid_spec = pltpu.PrefetchScalarGridSpec(
        num_scalar_prefetch=2,
        grid=(n_experts, n_tiles_max),
        in_specs=[
            pl.BlockSpec((BM, d_exp), lambda e, j, st, fl: (st[e * n_tiles_max + j], 0)),
            pl.BlockSpec((1, d_exp, d), lambda e, j, st, fl: (e, 0, 0)),
        ],
        out_specs=pl.BlockSpec((BM, d),
                               lambda e, j, st, fl: (st[e * n_tiles_max + j], 0)),
        scratch_shapes=[pltpu.VMEM((d_exp, d), jnp.bfloat16)],
    )
    return pl.pallas_call(
        functools.partial(_k2_body, n_tiles_max=n_tiles_max),
        grid_spec=grid_spec,
        out_shape=jax.ShapeDtypeStruct((m_pad, d), jnp.float32),
    )(steptile, flags, h, W_down)


def _combine_body(tok_w, chunk, nc, n_tok, x_hbm, y_hbm, pos_hbm, out_hbm,
                  p0_v, p1_v, y0, y1, xb, ob, sem, sem2):
    c = lax.axis_index("c")
    s = lax.axis_index("s")
    wid = s * nc + c
    d = xb.shape[1]
    nv = d // L

    for ci in range(tok_w // chunk):
        t0 = wid * tok_w + ci * chunk
        pltpu.sync_copy(pos_hbm.at[pl.ds(t0, chunk)], p0_v)
        pltpu.sync_copy(pos_hbm.at[pl.ds(n_tok + t0, chunk)], p1_v)
        cp0 = pltpu.async_copy(y_hbm.at[p0_v], y0, sem)
        cp1 = pltpu.async_copy(y_hbm.at[p1_v], y1, sem2)
        pltpu.sync_copy(x_hbm.at[pl.ds(t0, chunk)], xb)
        cp0.wait()
        cp1.wait()

        def row(i, _):
            for u in range(4):
                b = i * 4 + u
                r = b >> 6
                col = (b & (nv - 1)) * L
                ob[r, pl.ds(col, L)] = (xb[r, pl.ds(col, L)]
                                        + y0[r, pl.ds(col, L)]
                                        + y1[r, pl.ds(col, L)])
            return _

        lax.fori_loop(0, chunk * nv // 4, row, 0)
        pltpu.sync_copy(ob, out_hbm.at[pl.ds(t0, chunk)])


def _combine(x2, y, pos):
    t, d = x2.shape
    mesh = plsc.VectorSubcoreMesh(core_axis_name="c", subcore_axis_name="s", num_cores=2, num_subcores=16)
    nw = mesh.num_cores * mesh.num_subcores
    tok_w = t // nw
    chunk = 16
    f = pl.kernel(
        functools.partial(_combine_body, tok_w, chunk, mesh.num_cores, t),
        out_type=jax.ShapeDtypeStruct((t, d), jnp.float32),
        mesh=mesh,
        scratch_types=[
            pltpu.VMEM((chunk,), jnp.int32),
            pltpu.VMEM((chunk,), jnp.int32),
            pltpu.VMEM((chunk, d), jnp.float32),
            pltpu.VMEM((chunk, d), jnp.float32),
            pltpu.VMEM((chunk, d), jnp.float32),
            pltpu.VMEM((chunk, d), jnp.float32),
            pltpu.SemaphoreType.DMA,
            pltpu.SemaphoreType.DMA,
        ],
    )
    return f(x2, y, pos)


def kernel(x, norm_w, W_r, W_gate, W_up, W_down):
    orig_shape = x.shape
    d = x.shape[-1]
    t = x.size // d
    n_experts = W_gate.shape[0]
    n_tiles_max = t // BM
    m_pad = 2 * t + n_experts * BM + 2 * BM
    dump_tile = m_pad // BM - 2

    x2 = x.reshape(t, d)
    nw2 = norm_w.reshape(1, d)
    xn, ei, wp = _router(x2, nw2, W_r)
    pos, tok, wsort, steptile, flags = _sort(
        ei.reshape(2 * t), wp.reshape(2 * t), m_pad, n_experts,
        n_tiles_max, dump_tile)
    xpad = _gather(tok, xn, m_pad)
    h = _k1(steptile, flags, xpad, W_gate, W_up, wsort.reshape(m_pad, 1),
            n_tiles_max)
    y = _k2(steptile, flags, h, W_down, n_tiles_max)
    out = _combine(x2, y, pos)
    return out.reshape(orig_shape)


# speedup vs baseline: 1.1544x; 1.1303x over previous
import functools

import jax
import jax.numpy as jnp
from jax import lax
from jax.experimental import pallas as pl
from jax.experimental.pallas import tpu as pltpu
from jax.experimental.pallas import tpu_sc as plsc

BM = 128
L = 16


def _router_body(x_ref, nw_ref, wr_ref, xn_ref, ei_ref, wp_ref, *, n_experts):
    x = x_ref[...]
    var = jnp.mean(jnp.square(x), axis=-1, keepdims=True)
    xn = x * jax.lax.rsqrt(var + 1e-6) * nw_ref[...]
    xn_ref[...] = xn
    logits = jnp.dot(xn, wr_ref[...], preferred_element_type=jnp.float32)
    probs = jax.nn.softmax(logits, axis=-1)
    lane = jax.lax.broadcasted_iota(jnp.int32, probs.shape, 1)
    m1 = jnp.max(probs, axis=-1, keepdims=True)
    i1 = jnp.min(jnp.where(probs == m1, lane, n_experts), axis=-1, keepdims=True)
    pm = jnp.where(lane == i1, -jnp.inf, probs)
    m2 = jnp.max(pm, axis=-1, keepdims=True)
    i2 = jnp.min(jnp.where(pm == m2, lane, n_experts), axis=-1, keepdims=True)
    ei_ref[...] = jnp.concatenate([i1, i2], axis=1)
    wp_ref[...] = jnp.concatenate([m1, m2], axis=1)


def _router(x2, nw2, W_r):
    t, d = x2.shape
    e = W_r.shape[1]
    return pl.pallas_call(
        functools.partial(_router_body, n_experts=e),
        grid=(1,),
        in_specs=[
            pl.BlockSpec((t, d), lambda i: (0, 0)),
            pl.BlockSpec((1, d), lambda i: (0, 0)),
            pl.BlockSpec((d, e), lambda i: (0, 0)),
        ],
        out_specs=[
            pl.BlockSpec((t, d), lambda i: (0, 0)),
            pl.BlockSpec((t, 2), lambda i: (0, 0)),
            pl.BlockSpec((t, 2), lambda i: (0, 0)),
        ],
        out_shape=[
            jax.ShapeDtypeStruct((t, d), jnp.float32),
            jax.ShapeDtypeStruct((t, 2), jnp.int32),
            jax.ShapeDtypeStruct((t, 2), jnp.float32),
        ],
    )(x2, nw2, W_r)


def _lanes():
    return lax.iota(jnp.int32, L)


def _g(x, idx):
    return x.at[idx].get(mode="promise_in_bounds")


def _bsum(x):
    lanes = _lanes()
    for k in (1, 2, 4, 8):
        x = x + _g(x, lanes ^ k)
    return x


def _prefix_incl(x):
    lanes = _lanes()
    for k in (1, 2, 4, 8):
        sh = _g(x, jnp.maximum(lanes - k, 0))
        x = x + jnp.where(lanes >= k, sh, 0)
    return x


def _splat(x, e):
    return _g(x, jnp.full((L,), e, jnp.int32))


def _sort_body(n_pairs, m_pad, n_experts, n_tiles_max, dump_tile,
               eid_hbm,
               pos_hbm, tok_hbm, st_hbm, fl_hbm,
               eid_v, pos_v, tok_v, myhist, histall, hist_sh,
               zvi, stv, flv, sem):
    c = lax.axis_index("c")
    s = lax.axis_index("s")
    per_w = n_pairs // L
    pad_w = m_pad // L
    n_v = per_w // L
    lanes = lax.iota(jnp.int32, L)
    ones = jnp.ones((L,), jnp.int32)
    zeros = jnp.zeros((L,), jnp.int32)

    def _vcount(v, acc):
        lo = jnp.where(v < 4, jnp.where(v == 0, 1, 0) + jnp.where(v == 1, 1 << 8, 0)
                       + jnp.where(v == 2, 1 << 16, 0) + jnp.where(v == 3, 1 << 24, 0), 0)
        hi = jnp.where(v >= 4, jnp.where(v == 4, 1, 0) + jnp.where(v == 5, 1 << 8, 0)
                       + jnp.where(v == 6, 1 << 16, 0) + jnp.where(v == 7, 1 << 24, 0), 0)
        lo_t = _bsum(lo)
        hi_t = _bsum(hi)
        sel = jnp.where(lanes < 4, lo_t, hi_t)
        sh = (lanes & 3) << 3
        cnt = jax.lax.shift_right_logical(sel, sh) & 255
        return acc + jnp.where(lanes < n_experts, cnt, 0)

    @pl.when(c == 0)
    def _():
        for j in range(pad_w // L):
            zvi[pl.ds(j * L, L)] = zeros
        pltpu.sync_copy(zvi, tok_hbm.at[pl.ds(s * pad_w, pad_w)])

        pltpu.sync_copy(eid_hbm.at[pl.ds(s * per_w, per_w)], eid_v)

        hist = zeros
        for j in range(n_v):
            hist = _vcount(eid_v[pl.ds(j * L, L)], hist)
        myhist[...] = hist
        pltpu.sync_copy(myhist, hist_sh.at[s])
        plsc.subcore_barrier()

        pltpu.sync_copy(hist_sh, histall)

        totals = zeros
        for j in range(L):
            totals = totals + histall[j, :]

        def t_body(j, acc):
            return acc + histall[j, :]
        before = lax.fori_loop(0, s, t_body, zeros)
        padded = ((totals + (BM - 1)) >> 7) << 7
        incl = _prefix_incl(padded)
        base = incl - padded

        @pl.when(s == 0)
        def _():
            base_t = base >> 7
            nt_t = padded >> 7
            for e in range(n_experts):
                bt = _splat(base_t, e)
                nt = _splat(nt_t, e)
                m = bt + jnp.minimum(lanes, jnp.maximum(nt - 1, 0))
                flag = jnp.where(lanes < nt, 1, 0)
                m = jnp.where(flag == 1, jnp.minimum(m, dump_tile), dump_tile)
                stv[pl.ds(e * L, L)] = m
                flv[pl.ds(e * L, L)] = flag
            pltpu.sync_copy(stv, st_hbm)
            pltpu.sync_copy(flv, fl_hbm)

        p_base = s * per_w
        n_tok = n_pairs // 2

        off = base + before
        for j in range(n_v):
            v = eid_v[pl.ds(j * L, L)]
            off_l = _g(off, v)
            rank = zeros
            for k in range(1, L):
                sh = _g(v, jnp.maximum(lanes - k, 0))
                rank = rank + jnp.where((sh == v) & (lanes >= k), 1, 0)
            pos_v[pl.ds(j * L, L)] = off_l + rank
            p_vec = p_base + j * L + lanes
            tok_v[pl.ds(j * L, L)] = p_vec >> 1
            off = _vcount(v, off)

        pltpu.sync_copy(pos_v, pos_hbm.at[pl.ds(p_base, per_w)])
        pltpu.async_copy(tok_v, tok_hbm.at[pos_v], sem).wait()


def _sort(eflat, m_pad, n_experts, n_tiles_max, dump_tile):
    n_pairs = eflat.shape[0]
    per_w = n_pairs // L
    pad_w = m_pad // L
    n_steps = n_experts * n_tiles_max
    mesh = plsc.VectorSubcoreMesh(core_axis_name="c", subcore_axis_name="s", num_cores=2, num_subcores=16)
    f = pl.kernel(
        functools.partial(_sort_body, n_pairs, m_pad, n_experts, n_tiles_max,
                          dump_tile),
        out_type=[
            jax.ShapeDtypeStruct((n_pairs,), jnp.int32),
            jax.ShapeDtypeStruct((m_pad,), jnp.int32),
            jax.ShapeDtypeStruct((n_steps,), jnp.int32),
            jax.ShapeDtypeStruct((n_steps,), jnp.int32),
        ],
        mesh=mesh,
        scratch_types=[
            pltpu.VMEM((per_w,), jnp.int32),
            pltpu.VMEM((per_w,), jnp.int32),
            pltpu.VMEM((per_w,), jnp.int32),
            pltpu.VMEM((L,), jnp.int32),
            pltpu.VMEM((L, L), jnp.int32),
            pltpu.VMEM_SHARED((L, L), jnp.int32),
            pltpu.VMEM((pad_w,), jnp.int32),
            pltpu.VMEM((n_steps,), jnp.int32),
            pltpu.VMEM((n_steps,), jnp.int32),
            pltpu.SemaphoreType.DMA,
        ],
    )
    return f(eflat)


def _gather_body(rows_w, chunk, nc, depth, tok_hbm, xn_hbm, xp_hbm, *scr):
    c = lax.axis_index("c")
    s = lax.axis_index("s")
    wid = s * nc + c
    base = wid * rows_w
    nch = rows_w // chunk
    idx = scr[:depth]
    buf = scr[depth:2 * depth]
    sem = scr[2 * depth:3 * depth]

    cps = {}
    for k in range(min(depth, nch)):
        pltpu.sync_copy(tok_hbm.at[pl.ds(base + k * chunk, chunk)], idx[k])
        cps[k] = pltpu.async_copy(xn_hbm.at[idx[k]], buf[k], sem[k])
    for ci in range(nch):
        b = ci % depth
        cps[ci].wait()
        pltpu.sync_copy(buf[b], xp_hbm.at[pl.ds(base + ci * chunk, chunk)])
        nxt = ci + depth
        if nxt < nch:
            pltpu.sync_copy(tok_hbm.at[pl.ds(base + nxt * chunk, chunk)], idx[b])
            cps[nxt] = pltpu.async_copy(xn_hbm.at[idx[b]], buf[b], sem[b])


def _gather(tok, xn, m_pad):
    t, d = xn.shape
    mesh = plsc.VectorSubcoreMesh(core_axis_name="c", subcore_axis_name="s", num_cores=2, num_subcores=16)
    nw = mesh.num_cores * mesh.num_subcores
    rows_w = m_pad // nw
    chunk = 24
    depth = 4
    f = pl.kernel(
        functools.partial(_gather_body, rows_w, chunk, mesh.num_cores, depth),
        out_type=jax.ShapeDtypeStruct((m_pad, d), jnp.float32),
        mesh=mesh,
        scratch_types=(
            [pltpu.VMEM((chunk,), jnp.int32) for _ in range(depth)]
            + [pltpu.VMEM((chunk, d), jnp.float32) for _ in range(depth)]
            + [pltpu.SemaphoreType.DMA for _ in range(depth)]
        ),
    )
    return f(tok, xn)


def _k1_body(st_ref, fl_ref, x_ref, wg_ref, wu_ref, h_ref,
             wg_bf, wu_bf, *, n_tiles_max):
    e = pl.program_id(0)
    j = pl.program_id(1)
    flag = fl_ref[e * n_tiles_max + j] == 1

    @pl.when(flag & (j == 0))
    def _():
        wg_bf[...] = wg_ref[0].astype(jnp.bfloat16)
        wu_bf[...] = wu_ref[0].astype(jnp.bfloat16)

    @pl.when(flag)
    def _():
        xb = x_ref[...].astype(jnp.bfloat16)
        gate = jnp.dot(xb, wg_bf[...], preferred_element_type=jnp.float32)
        up = jnp.dot(xb, wu_bf[...], preferred_element_type=jnp.float32)
        h = gate * jax.lax.logistic(gate) * up
        h_ref[...] = h.astype(jnp.bfloat16)


def _k1(steptile, flags, xpad, W_gate, W_up, n_tiles_max):
    m_pad, d = xpad.shape
    n_experts, _, d_exp = W_gate.shape
    grid_spec = pltpu.PrefetchScalarGridSpec(
        num_scalar_prefetch=2,
        grid=(n_experts, n_tiles_max),
        in_specs=[
            pl.BlockSpec((BM, d), lambda e, j, st, fl: (st[e * n_tiles_max + j], 0)),
            pl.BlockSpec((1, d, d_exp), lambda e, j, st, fl: (e, 0, 0)),
            pl.BlockSpec((1, d, d_exp), lambda e, j, st, fl: (e, 0, 0)),
        ],
        out_specs=pl.BlockSpec((BM, d_exp),
                               lambda e, j, st, fl: (st[e * n_tiles_max + j], 0)),
        scratch_shapes=[
            pltpu.VMEM((d, d_exp), jnp.bfloat16),
            pltpu.VMEM((d, d_exp), jnp.bfloat16),
        ],
    )
    return pl.pallas_call(
        functools.partial(_k1_body, n_tiles_max=n_tiles_max),
        grid_spec=grid_spec,
        out_shape=jax.ShapeDtypeStruct((m_pad, d_exp), jnp.bfloat16),
    )(steptile, flags, xpad, W_gate, W_up)


def _k2_body(st_ref, fl_ref, h_ref, wd_ref, y_ref, wd_bf, *, n_tiles_max):
    e = pl.program_id(0)
    j = pl.program_id(1)
    flag = fl_ref[e * n_tiles_max + j] == 1

    @pl.when(flag & (j == 0))
    def _():
        wd_bf[...] = wd_ref[0].astype(jnp.bfloat16)

    @pl.when(flag)
    def _():
        y_ref[...] = jnp.dot(h_ref[...], wd_bf[...],
                             preferred_element_type=jnp.float32)


def _k2(steptile, flags, h, W_down, n_tiles_max):
    m_pad, d_exp = h.shape
    n_experts, _, d = W_down.shape
    grid_spec = pltpu.PrefetchScalarGridSpec(
        num_scalar_prefetch=2,
        grid=(n_experts, n_tiles_max),
        in_specs=[
            pl.BlockSpec((BM, d_exp), lambda e, j, st, fl: (st[e * n_tiles_max + j], 0)),
            pl.BlockSpec((1, d_exp, d), lambda e, j, st, fl: (e, 0, 0)),
        ],
        out_specs=pl.BlockSpec((BM, d),
                               lambda e, j, st, fl: (st[e * n_tiles_max + j], 0)),
        scratch_shapes=[pltpu.VMEM((d_exp, d), jnp.bfloat16)],
    )
    return pl.pallas_call(
        functools.partial(_k2_body, n_tiles_max=n_tiles_max),
        grid_spec=grid_spec,
        out_shape=jax.ShapeDtypeStruct((m_pad, d), jnp.float32),
    )(steptile, flags, h, W_down)


def _combine_body(tok_w, chunk, nc, x_hbm, y_hbm, pos_hbm, wp_hbm, out_hbm,
                  posb, wb, p0_v, p1_v, y0a, y1a, y0b, y1b, xb, ob, s0, s1):
    c = lax.axis_index("c")
    s = lax.axis_index("s")
    wid = s * nc + c
    lanes = _lanes()
    d = xb.shape[1]
    nv = d // L
    nch = tok_w // chunk
    ybufs = ((y0a, y1a), (y0b, y1b))

    def deinter(bufref, add):
        a = bufref[pl.ds(0, L)]
        b = bufref[pl.ds(L, L)]
        il = lanes * 2 + add
        return jnp.where(lanes < 8, _g(a, jnp.minimum(il, L - 1)),
                         _g(b, jnp.maximum(il - L, 0)))

    def issue(ci, pair):
        t0 = wid * tok_w + ci * chunk
        pltpu.sync_copy(pos_hbm.at[pl.ds(2 * t0, 2 * chunk)], posb)
        p0_v[...] = deinter(posb, 0)
        p1_v[...] = deinter(posb, 1)
        y0, y1 = ybufs[pair]
        return (pltpu.async_copy(y_hbm.at[p0_v], y0, s0),
                pltpu.async_copy(y_hbm.at[p1_v], y1, s1))

    cps = issue(0, 0)
    for ci in range(nch):
        pair = ci & 1
        t0 = wid * tok_w + ci * chunk
        pltpu.sync_copy(wp_hbm.at[pl.ds(2 * t0, 2 * chunk)], wb)
        w0 = deinter(wb, 0)
        w1 = deinter(wb, 1)
        pltpu.sync_copy(x_hbm.at[pl.ds(t0, chunk)], xb)
        cps[0].wait()
        cps[1].wait()
        nxt = ci + 1
        if nxt < nch:
            nxt_cps = issue(nxt, nxt & 1)
        y0, y1 = ybufs[pair]

        def row(i, _):
            r = i >> 4
            w0r = _g(w0, jnp.full((L,), 0, jnp.int32) + r)
            w1r = _g(w1, jnp.full((L,), 0, jnp.int32) + r)
            for u in range(4):
                b = (i & 15) * 4 + u
                col = b * L
                ob[r, pl.ds(col, L)] = (xb[r, pl.ds(col, L)]
                                        + w0r * y0[r, pl.ds(col, L)]
                                        + w1r * y1[r, pl.ds(col, L)])
            return _

        lax.fori_loop(0, chunk * L, row, 0)
        pltpu.sync_copy(ob, out_hbm.at[pl.ds(t0, chunk)])
        if nxt < nch:
            cps = nxt_cps


def _combine(x2, y, pos, wp):
    t, d = x2.shape
    mesh = plsc.VectorSubcoreMesh(core_axis_name="c", subcore_axis_name="s", num_cores=2, num_subcores=16)
    nw = mesh.num_cores * mesh.num_subcores
    tok_w = t // nw
    chunk = 16
    f = pl.kernel(
        functools.partial(_combine_body, tok_w, chunk, mesh.num_cores),
        out_type=jax.ShapeDtypeStruct((t, d), jnp.float32),
        mesh=mesh,
        scratch_types=[
            pltpu.VMEM((2 * chunk,), jnp.int32),
            pltpu.VMEM((2 * chunk,), jnp.float32),
            pltpu.VMEM((L,), jnp.int32),
            pltpu.VMEM((L,), jnp.int32),
            pltpu.VMEM((chunk, d), jnp.float32),
            pltpu.VMEM((chunk, d), jnp.float32),
            pltpu.VMEM((chunk, d), jnp.float32),
            pltpu.VMEM((chunk, d), jnp.float32),
            pltpu.VMEM((chunk, d), jnp.float32),
            pltpu.VMEM((chunk, d), jnp.float32),
            pltpu.SemaphoreType.DMA,
            pltpu.SemaphoreType.DMA,
        ],
    )
    return f(x2, y, pos, wp)


def kernel(x, norm_w, W_r, W_gate, W_up, W_down):
    orig_shape = x.shape
    d = x.shape[-1]
    t = x.size // d
    n_experts = W_gate.shape[0]
    n_tiles_max = t // BM
    m_pad = 2 * t + n_experts * BM + 2 * BM
    dump_tile = m_pad // BM - 2

    x2 = x.reshape(t, d)
    nw2 = norm_w.reshape(1, d)
    xn, ei, wp = _router(x2, nw2, W_r)
    pos, tok, steptile, flags = _sort(
        ei.reshape(2 * t), m_pad, n_experts, n_tiles_max, dump_tile)
    xpad = _gather(tok, xn, m_pad)
    h = _k1(steptile, flags, xpad, W_gate, W_up, n_tiles_max)
    y = _k2(steptile, flags, h, W_down, n_tiles_max)
    out = _combine(x2, y, pos, wp.reshape(2 * t))
    return out.reshape(orig_shape)
